# baseline (speedup 1.0000x reference)
import functools

import jax
import jax.numpy as jnp
from jax import lax
from jax.experimental import pallas as pl
from jax.experimental.pallas import tpu as pltpu
from jax.experimental.pallas import tpu_sc as plsc

B, S = 16384, 200
V, D = 1_000_000, 32
NC, NS = 2, 16
NW = NC * NS
BT = B // 128
TPW = BT // NW
DT = D // 8

_mesh = plsc.VectorSubcoreMesh(core_axis_name="c", subcore_axis_name="s")


@functools.partial(
    pl.kernel,
    out_type=jax.ShapeDtypeStruct((S, DT, BT, 8, 8, 16), jnp.float32),
    mesh=_mesh,
    scratch_types=[
        pltpu.VMEM((TPW * 128,), jnp.int32),
        pltpu.VMEM((TPW * 128,), jnp.int32),
        pltpu.VMEM((128, D), jnp.float32),
        pltpu.VMEM((128, D), jnp.float32),
        pltpu.VMEM((DT, 8, 8, 16), jnp.float32),
        pltpu.VMEM((DT, 8, 8, 16), jnp.float32),
        pltpu.SemaphoreType.DMA,
        pltpu.SemaphoreType.DMA,
        pltpu.SemaphoreType.DMA,
        pltpu.SemaphoreType.DMA,
        pltpu.SemaphoreType.DMA,
        pltpu.SemaphoreType.DMA,
    ],
    compiler_params=pltpu.CompilerParams(use_tc_tiling_on_sc=False, needs_layout_passes=False),
)
def _gather(table_hbm, seqs_t_hbm, out_hbm,
            idx0, idx1, rows0, rows1, t0, t1,
            is0, is1, gs0, gs1, ws0, ws1):
    idx_v = (idx0, idx1)
    rows_v = (rows0, rows1)
    tbuf = (t0, t1)
    isem = (is0, is1)
    gsem = (gs0, gs1)
    wsem = (ws0, ws1)

    wid = lax.axis_index("s") * NC + lax.axis_index("c")
    tj0 = wid * TPW
    col0 = tj0 * 128

    bvecs = [lax.iota(jnp.int32, 16) + 16 * k for k in range(8)]
    dvecs = [jnp.full((16,), d, jnp.int32) for d in range(D)]

    def idx_src(s):
        return seqs_t_hbm.at[s, pl.ds(col0, TPW * 128)]

    def gather_unit(sb, tjr, b):
        return pltpu.async_copy(
            table_hbm.at[idx_v[sb].at[pl.ds(tjr * 128, 128)]],
            rows_v[b],
            gsem[b],
        )

    def wait_gather(b):
        pltpu.make_async_copy(
            table_hbm.at[idx_v[0].at[pl.ds(0, 128)]], rows_v[b], gsem[b]
        ).wait()

    pltpu.async_copy(idx_src(0), idx_v[0], isem[0])
    pltpu.async_copy(idx_src(1), idx_v[1], isem[1])
    pltpu.make_async_copy(idx_src(0), idx_v[0], isem[0]).wait()
    gather_unit(0, 0, 0)
    gather_unit(0, 1, 1)

    def pos_pair(sq, carry):
        for sp in range(2):
            s = sq * 2 + sp
            body(s, sp)
        return carry

    def body(s, sp):
        for tjr in range(TPW):
            b = tjr & 1
            u = s * TPW + tjr

            wait_gather(b)

            if tjr == TPW - 1:
                @pl.when(s + 2 < S)
                def _():
                    pltpu.async_copy(idx_src(s + 2), idx_v[sp], isem[sp])

            @pl.when(u > 1)
            def _():
                for ti in range(DT):
                    pltpu.make_async_copy(
                        tbuf[b].at[ti], out_hbm.at[0, ti, 0], wsem[b]
                    ).wait()

            rv, tb = rows_v[b], tbuf[b]
            for ti in range(DT):
                for di in range(8):
                    d = ti * 8 + di
                    vals = [
                        plsc.load_gather(rv, [bvecs[k], dvecs[d]])
                        for k in range(8)
                    ]
                    for k in range(8):
                        tb[ti, di, k, :] = vals[k]

            if tjr + 2 < TPW:
                gather_unit(sp, tjr + 2, b)
            else:
                @pl.when(s + 1 < S)
                def _():
                    t2 = tjr + 2 - TPW
                    if t2 == 0:
                        pltpu.make_async_copy(
                            idx_src(s + 1), idx_v[1 - sp], isem[1 - sp]
                        ).wait()
                    gather_unit(1 - sp, t2, b)

            tj = tj0 + tjr
            for ti in range(DT):
                pltpu.async_copy(
                    tbuf[b].at[ti], out_hbm.at[s, ti, tj], wsem[b]
                )

    lax.fori_loop(0, S // 2, pos_pair, 0)

    for b in range(2):
        for ti in range(DT):
            pltpu.make_async_copy(
                tbuf[b].at[ti], out_hbm.at[0, ti, 0], wsem[b]
            ).wait()


def kernel(seqs, species, table):
    del species
    seqs_t = jnp.transpose(seqs).astype(jnp.int32)
    und6 = _gather(table, seqs_t)
    return und6.transpose((2, 4, 5, 0, 1, 3)).reshape(B, S, D)


# speedup vs baseline: 2.6574x; 2.6574x over previous
import functools

import jax
import jax.numpy as jnp
from jax import lax
from jax.experimental import pallas as pl
from jax.experimental.pallas import tpu as pltpu
from jax.experimental.pallas import tpu_sc as plsc

B, S = 16384, 200
V, D = 1_000_000, 32
N = B * S
NC, NS = 2, 16
NW = NC * NS
NPW = N // NW
C = 1024
NCHUNK = NPW // C
NBUF = 2
NSUPER = NCHUNK // NBUF

_mesh = plsc.VectorSubcoreMesh(core_axis_name="c", subcore_axis_name="s")


@functools.partial(
    pl.kernel,
    out_type=jax.ShapeDtypeStruct((N, D), jnp.float32),
    mesh=_mesh,
    scratch_types=[
        pltpu.VMEM((C,), jnp.int32),
        pltpu.VMEM((C,), jnp.int32),
        pltpu.VMEM((C, D), jnp.float32),
        pltpu.VMEM((C, D), jnp.float32),
        pltpu.SemaphoreType.DMA,
        pltpu.SemaphoreType.DMA,
        pltpu.SemaphoreType.DMA,
        pltpu.SemaphoreType.DMA,
        pltpu.SemaphoreType.DMA,
        pltpu.SemaphoreType.DMA,
    ],
    compiler_params=pltpu.CompilerParams(use_tc_tiling_on_sc=False),
)
def _gather(table_hbm, idx_hbm, out_hbm,
            idx0, idx1, rows0, rows1, is0, is1, gs0, gs1, os0, os1):
    idx_v = (idx0, idx1)
    rows_v = (rows0, rows1)
    isem = (is0, is1)
    gsem = (gs0, gs1)
    osem = (os0, os1)

    wid = lax.axis_index("s") * NC + lax.axis_index("c")
    base = wid * NPW

    def idx_src(ci):
        return idx_hbm.at[pl.ds(base + ci * C, C)]

    def out_dst(ci):
        return out_hbm.at[pl.ds(base + ci * C, C)]

    for b in range(NBUF):
        pltpu.async_copy(idx_src(b), idx_v[b], isem[b])

    def super_chunk(g, carry):
        for b in range(NBUF):
            ci = g * NBUF + b
            pltpu.make_async_copy(idx_src(ci), idx_v[b], isem[b]).wait()

            @pl.when(g > 0)
            def _():
                pltpu.make_async_copy(rows_v[b], out_dst(ci), osem[b]).wait()

            pltpu.async_copy(
                table_hbm.at[idx_v[b]], rows_v[b], gsem[b]
            ).wait()

            pltpu.async_copy(rows_v[b], out_dst(ci), osem[b])

            @pl.when(g < NSUPER - 1)
            def _():
                pltpu.async_copy(idx_src(ci + NBUF), idx_v[b], isem[b])
        return carry

    lax.fori_loop(0, NSUPER, super_chunk, 0)

    for b in range(NBUF):
        pltpu.make_async_copy(
            rows_v[b], out_dst(NCHUNK - NBUF + b), osem[b]
        ).wait()


def kernel(seqs, species, table):
    del species
    idx_flat = seqs.reshape(-1).astype(jnp.int32)
    out = _gather(table, idx_flat)
    return out.reshape(B, S, D)


# speedup vs baseline: 2.8193x; 1.0609x over previous
import functools

import jax
import jax.numpy as jnp
from jax import lax
from jax.experimental import pallas as pl
from jax.experimental.pallas import tpu as pltpu
from jax.experimental.pallas import tpu_sc as plsc

B, S = 16384, 200
V, D = 1_000_000, 32
NC, NS = 2, 16
NW = NC * NS
BPW = B // NW

_mesh = plsc.VectorSubcoreMesh(core_axis_name="c", subcore_axis_name="s")


@functools.partial(
    pl.kernel,
    out_type=jax.ShapeDtypeStruct((S, B, D), jnp.float32),
    mesh=_mesh,
    scratch_types=[
        pltpu.VMEM((BPW,), jnp.int32),
        pltpu.VMEM((BPW,), jnp.int32),
        pltpu.VMEM((BPW, D), jnp.float32),
        pltpu.VMEM((BPW, D), jnp.float32),
        pltpu.SemaphoreType.DMA,
        pltpu.SemaphoreType.DMA,
        pltpu.SemaphoreType.DMA,
        pltpu.SemaphoreType.DMA,
        pltpu.SemaphoreType.DMA,
        pltpu.SemaphoreType.DMA,
    ],
    compiler_params=pltpu.CompilerParams(use_tc_tiling_on_sc=False),
)
def _gather(table_hbm, seqs_t_hbm, out_hbm,
            idx0, idx1, rows0, rows1, is0, is1, gs0, gs1, os0, os1):
    idx_v = (idx0, idx1)
    rows_v = (rows0, rows1)
    isem = (is0, is1)
    gsem = (gs0, gs1)
    osem = (os0, os1)

    wid = lax.axis_index("s") * NC + lax.axis_index("c")
    b0 = wid * BPW

    def idx_src(s):
        return seqs_t_hbm.at[s, pl.ds(b0, BPW)]

    def out_dst(s):
        return out_hbm.at[s, pl.ds(b0, BPW)]

    for p in range(2):
        pltpu.async_copy(idx_src(p), idx_v[p], isem[p])

    def pos_pair(sq, carry):
        for p in range(2):
            s = sq * 2 + p
            pltpu.make_async_copy(idx_src(s), idx_v[p], isem[p]).wait()

            @pl.when(sq > 0)
            def _():
                pltpu.make_async_copy(rows_v[p], out_dst(s), osem[p]).wait()

            pltpu.async_copy(
                table_hbm.at[idx_v[p]], rows_v[p], gsem[p]
            ).wait()

            pltpu.async_copy(rows_v[p], out_dst(s), osem[p])

            @pl.when(s + 2 < S)
            def _():
                pltpu.async_copy(idx_src(s + 2), idx_v[p], isem[p])
        return carry

    lax.fori_loop(0, S // 2, pos_pair, 0)

    for p in range(2):
        pltpu.make_async_copy(
            rows_v[p], out_dst(S - 2 + p), osem[p]
        ).wait()


def kernel(seqs, species, table):
    del species
    seqs_t = jnp.transpose(seqs).astype(jnp.int32)
    und_t = _gather(table, seqs_t)
    return und_t.transpose((1, 0, 2))
